# Initial kernel scaffold; baseline (speedup 1.0000x reference)
import jax
import jax.numpy as jnp
from jax.experimental import pallas as pl


def kernel(anchor_bboxes, num_anchors_list, gt_labels, gt_bboxes, pad_gt_mask, bg_index):
    raise NotImplementedError("write your pallas kernel here")


# speedup vs baseline: 10.6638x; 10.6638x over previous
import functools

import jax
import jax.numpy as jnp
import numpy as np
from jax.experimental import pallas as pl
from jax.experimental.pallas import tpu as pltpu

_TOPK = 9
_NUM_CLASSES = 80
_EPS = 1e-09


def _atss_body(bg_ref, anchors_ref, gt_boxes_ref, gt_labels_ref, mask_ref,
               labels_out, boxes_out, scores_out, *, level_sizes, M, A):
    f32 = jnp.float32
    ax1 = anchors_ref[0:1, :]
    ay1 = anchors_ref[1:2, :]
    ax2 = anchors_ref[2:3, :]
    ay2 = anchors_ref[3:4, :]
    acx = (ax1 + ax2) / 2.0
    acy = (ay1 + ay2) / 2.0
    area_a = (ax2 - ax1) * (ay2 - ay1)

    gb = gt_boxes_ref[0]
    gx1 = gb[:, 0:1]
    gy1 = gb[:, 1:2]
    gx2 = gb[:, 2:3]
    gy2 = gb[:, 3:4]
    area_g = (gx2 - gx1) * (gy2 - gy1)

    ltx = jnp.maximum(gx1, ax1)
    lty = jnp.maximum(gy1, ay1)
    rbx = jnp.minimum(gx2, ax2)
    rby = jnp.minimum(gy2, ay2)
    iw = jnp.maximum(rbx - ltx, 0.0)
    ih = jnp.maximum(rby - lty, 0.0)
    inter = iw * ih
    union = area_g + area_a - inter
    iou = inter / (union + _EPS)

    gcx = (gx1 + gx2) / 2.0
    gcy = (gy1 + gy2) / 2.0
    dx = gcx - acx
    dy = gcy - acy
    dist = jnp.sqrt(dx * dx + dy * dy)

    sel_parts = []
    start = 0
    for na in level_sizes:
        work = dist[:, start:start + na]
        col = jax.lax.broadcasted_iota(jnp.int32, (M, na), 1)
        sel = jnp.zeros((M, na), dtype=jnp.bool_)
        for _ in range(_TOPK):
            idx = jnp.argmin(work, axis=1).reshape(M, 1)
            hit = col == idx
            sel = jnp.logical_or(sel, hit)
            work = jnp.where(hit, jnp.inf, work)
        sel_parts.append(sel.astype(f32))
        start += na
    sel_f = jnp.concatenate(sel_parts, axis=1)
    sel = sel_f > 0.0

    n_sel = float(_TOPK * len(level_sizes))
    cand = iou * sel_f
    mean = jnp.sum(cand, axis=1, keepdims=True) / n_sel
    dev = (iou - mean) * sel_f
    var = jnp.sum(dev * dev, axis=1, keepdims=True) / (n_sel - 1.0)
    thresh = mean + jnp.sqrt(var)

    d1 = acx - gx1
    d2 = acy - gy1
    d3 = gx2 - acx
    d4 = gy2 - acy
    min_d = jnp.minimum(jnp.minimum(d1, d2), jnp.minimum(d3, d4))
    in_gts = min_d > _EPS

    valid = mask_ref[0][:, 0:1] > 0.0
    pos = jnp.logical_and(jnp.logical_and(sel, cand > thresh),
                          jnp.logical_and(in_gts, valid))
    pos_f = pos.astype(f32)
    pos_sum = jnp.sum(pos_f, axis=0, keepdims=True)
    multi = pos_sum > 1.0
    assigned = pos_sum > 0.0

    miota = jax.lax.broadcasted_iota(jnp.int32, (M, A), 0)
    max_iou = jnp.max(iou, axis=0, keepdims=True)
    idx_iou = jnp.min(jnp.where(iou == max_iou, miota, M), axis=0,
                      keepdims=True)
    idx_pos = jnp.min(jnp.where(pos, miota, M), axis=0, keepdims=True)
    idx_pos = jnp.where(idx_pos == M, 0, idx_pos)
    assigned_idx = jnp.where(multi, idx_iou, idx_pos)

    oh = miota == assigned_idx
    oh_f = oh.astype(f32)
    boxes = jax.lax.dot_general(oh_f, gb, (((0,), (0,)), ((), ())),
                                preferred_element_type=f32)
    boxes_out[0] = boxes

    gl = gt_labels_ref[0][:, 0:1]
    label = jnp.sum(jnp.where(oh, gl, 0), axis=0, keepdims=True)
    labels_out[0, 0, :] = jnp.where(assigned, label, bg_ref[0])[0]

    ciota = jax.lax.broadcasted_iota(jnp.int32, (M, _NUM_CLASSES), 1)
    class_oh = (gl == ciota).astype(f32)
    oh_masked = jnp.logical_and(oh, assigned).astype(f32)
    scores = jax.lax.dot_general(oh_masked, class_oh, (((0,), (0,)), ((), ())),
                                 preferred_element_type=f32)
    scores_out[0] = scores


def kernel(anchor_bboxes, num_anchors_list, gt_labels, gt_bboxes, pad_gt_mask,
           bg_index):
    A = anchor_bboxes.shape[0]
    B, M = gt_bboxes.shape[0], gt_bboxes.shape[1]
    levels = len(num_anchors_list)
    denom = sum(4 ** (levels - 1 - i) for i in range(levels))
    unit = A // denom
    level_sizes = tuple(unit * 4 ** (levels - 1 - i) for i in range(levels))

    anchors_t = anchor_bboxes.T
    bg = jnp.asarray(bg_index, jnp.int32).reshape(1)
    gt_labels_i = gt_labels.astype(jnp.int32)

    body = functools.partial(_atss_body, level_sizes=level_sizes, M=M, A=A)
    labels3, boxes, scores = pl.pallas_call(
        body,
        grid=(B,),
        in_specs=[
            pl.BlockSpec(memory_space=pltpu.SMEM),
            pl.BlockSpec((4, A), lambda b: (0, 0)),
            pl.BlockSpec((1, M, 4), lambda b: (b, 0, 0)),
            pl.BlockSpec((1, M, 1), lambda b: (b, 0, 0)),
            pl.BlockSpec((1, M, 1), lambda b: (b, 0, 0)),
        ],
        out_specs=[
            pl.BlockSpec((1, 1, A), lambda b: (b, 0, 0)),
            pl.BlockSpec((1, A, 4), lambda b: (b, 0, 0)),
            pl.BlockSpec((1, A, _NUM_CLASSES), lambda b: (b, 0, 0)),
        ],
        out_shape=[
            jax.ShapeDtypeStruct((B, 1, A), jnp.int32),
            jax.ShapeDtypeStruct((B, A, 4), jnp.float32),
            jax.ShapeDtypeStruct((B, A, _NUM_CLASSES), jnp.float32),
        ],
    )(bg, anchors_t, gt_bboxes, gt_labels_i, pad_gt_mask)
    return labels3.reshape(B, A), boxes, scores


# speedup vs baseline: 10.6717x; 1.0007x over previous
import functools

import jax
import jax.numpy as jnp
import numpy as np
from jax.experimental import pallas as pl
from jax.experimental.pallas import tpu as pltpu

_TOPK = 9
_NUM_CLASSES = 80
_EPS = 1e-09


def _atss_body(bg_ref, anchors_ref, gt_boxes_ref, gt_labels_ref, mask_ref,
               labels_out, boxes_out, scores_out, *, level_sizes, M, A):
    f32 = jnp.float32
    ax1 = anchors_ref[0:1, :]
    ay1 = anchors_ref[1:2, :]
    ax2 = anchors_ref[2:3, :]
    ay2 = anchors_ref[3:4, :]
    acx = (ax1 + ax2) / 2.0
    acy = (ay1 + ay2) / 2.0
    area_a = (ax2 - ax1) * (ay2 - ay1)

    gb = gt_boxes_ref[0]
    gx1 = gb[:, 0:1]
    gy1 = gb[:, 1:2]
    gx2 = gb[:, 2:3]
    gy2 = gb[:, 3:4]
    area_g = (gx2 - gx1) * (gy2 - gy1)

    ltx = jnp.maximum(gx1, ax1)
    lty = jnp.maximum(gy1, ay1)
    rbx = jnp.minimum(gx2, ax2)
    rby = jnp.minimum(gy2, ay2)
    iw = jnp.maximum(rbx - ltx, 0.0)
    ih = jnp.maximum(rby - lty, 0.0)
    inter = iw * ih
    union = area_g + area_a - inter
    iou = inter / (union + _EPS)

    gcx = (gx1 + gx2) / 2.0
    gcy = (gy1 + gy2) / 2.0
    dx = gcx - acx
    dy = gcy - acy
    dist = jnp.sqrt(dx * dx + dy * dy)

    sel_parts = []
    start = 0
    for na in level_sizes:
        work = dist[:, start:start + na]
        col = jax.lax.broadcasted_iota(jnp.int32, (M, na), 1)
        sel = jnp.zeros((M, na), dtype=jnp.bool_)
        for _ in range(_TOPK):
            idx = jnp.argmin(work, axis=1).reshape(M, 1)
            hit = col == idx
            sel = jnp.logical_or(sel, hit)
            work = jnp.where(hit, jnp.inf, work)
        sel_parts.append(sel.astype(f32))
        start += na
    sel_f = jnp.concatenate(sel_parts, axis=1)
    sel = sel_f > 0.0

    n_sel = float(_TOPK * len(level_sizes))
    cand = iou * sel_f
    mean = jnp.sum(cand, axis=1, keepdims=True) / n_sel
    dev = (iou - mean) * sel_f
    var = jnp.sum(dev * dev, axis=1, keepdims=True) / (n_sel - 1.0)
    thresh = mean + jnp.sqrt(var)

    d1 = acx - gx1
    d2 = acy - gy1
    d3 = gx2 - acx
    d4 = gy2 - acy
    min_d = jnp.minimum(jnp.minimum(d1, d2), jnp.minimum(d3, d4))
    in_gts = min_d > _EPS

    valid = mask_ref[0][:, 0:1] > 0.0
    pos = jnp.logical_and(jnp.logical_and(sel, cand > thresh),
                          jnp.logical_and(in_gts, valid))
    pos_f = pos.astype(f32)
    pos_sum = jnp.sum(pos_f, axis=0, keepdims=True)
    multi = pos_sum > 1.0
    assigned = pos_sum > 0.0

    miota = jax.lax.broadcasted_iota(jnp.int32, (M, A), 0)
    max_iou = jnp.max(iou, axis=0, keepdims=True)
    idx_iou = jnp.min(jnp.where(iou == max_iou, miota, M), axis=0,
                      keepdims=True)
    idx_pos = jnp.min(jnp.where(pos, miota, M), axis=0, keepdims=True)
    idx_pos = jnp.where(idx_pos == M, 0, idx_pos)
    assigned_idx = jnp.where(multi, idx_iou, idx_pos)

    oh = miota == assigned_idx
    oh_f = oh.astype(f32)
    boxes = jax.lax.dot_general(oh_f, gb, (((0,), (0,)), ((), ())),
                                preferred_element_type=f32)
    boxes_out[0] = boxes

    gl = gt_labels_ref[0][:, 0:1]
    label = jnp.sum(jnp.where(oh, gl, 0), axis=0, keepdims=True)
    labels_out[0, 0, :] = jnp.where(assigned, label, bg_ref[0])[0]

    ciota = jax.lax.broadcasted_iota(jnp.int32, (M, _NUM_CLASSES), 1)
    class_oh = (gl == ciota).astype(f32)
    oh_masked = jnp.logical_and(oh, assigned).astype(f32)
    scores = jax.lax.dot_general(oh_masked, class_oh, (((0,), (0,)), ((), ())),
                                 preferred_element_type=f32)
    scores_out[0] = scores


def kernel(anchor_bboxes, num_anchors_list, gt_labels, gt_bboxes, pad_gt_mask,
           bg_index):
    A = anchor_bboxes.shape[0]
    B, M = gt_bboxes.shape[0], gt_bboxes.shape[1]
    levels = len(num_anchors_list)
    denom = sum(4 ** (levels - 1 - i) for i in range(levels))
    unit = A // denom
    level_sizes = tuple(unit * 4 ** (levels - 1 - i) for i in range(levels))

    anchors_t = anchor_bboxes.T
    bg = jnp.asarray(bg_index, jnp.int32).reshape(1)
    gt_labels_i = gt_labels.astype(jnp.int32)

    body = functools.partial(_atss_body, level_sizes=level_sizes, M=M, A=A)
    labels3, boxes, scores = pl.pallas_call(
        body,
        grid=(B,),
        in_specs=[
            pl.BlockSpec(memory_space=pltpu.SMEM),
            pl.BlockSpec((4, A), lambda b: (0, 0)),
            pl.BlockSpec((1, M, 4), lambda b: (b, 0, 0)),
            pl.BlockSpec((1, M, 1), lambda b: (b, 0, 0)),
            pl.BlockSpec((1, M, 1), lambda b: (b, 0, 0)),
        ],
        out_specs=[
            pl.BlockSpec((1, 1, A), lambda b: (b, 0, 0)),
            pl.BlockSpec((1, A, 4), lambda b: (b, 0, 0)),
            pl.BlockSpec((1, A, _NUM_CLASSES), lambda b: (b, 0, 0)),
        ],
        out_shape=[
            jax.ShapeDtypeStruct((B, 1, A), jnp.int32),
            jax.ShapeDtypeStruct((B, A, 4), jnp.float32),
            jax.ShapeDtypeStruct((B, A, _NUM_CLASSES), jnp.float32),
        ],
        compiler_params=pltpu.CompilerParams(
            dimension_semantics=("parallel",)),
    )(bg, anchors_t, gt_bboxes, gt_labels_i, pad_gt_mask)
    return labels3.reshape(B, A), boxes, scores
